# baseline (speedup 1.0000x reference)
import jax
import jax.numpy as jnp
from jax import lax
from jax.experimental import pallas as pl
from jax.experimental.pallas import tpu as pltpu
from jax.experimental.pallas import tpu_sc as plsc

N = 10000
NP = 10240
D = 128
DH = 64
E_TOT = 330000
NC = 2
NS = 16
K = 128
CPT = 20992
NCHUNK = CPT // K
NBUF = 4
E_PAD = NS * CPT
ROWS_PT = NP // NS
RCH = ROWS_PT // K
R_TC = 1024
GRID_TC = NP // R_TC


def _tc_common(xin, i, w_ref, as_ref, ad_ref, h_ref, asrc_ref, adst_ref,
               m_ref, smax_ref):
    h = jnp.dot(xin, w_ref[...], preferred_element_type=jnp.float32)
    h_ref[...] = h
    a_src = jnp.sum(h * as_ref[...], axis=1, keepdims=True)
    a_dst = jnp.sum(h * ad_ref[...], axis=1, keepdims=True)
    asrc_ref[...] = a_src
    adst_ref[...] = a_dst

    @pl.when(i == 0)
    def _():
        smax_ref[0] = -jnp.inf
        smax_ref[1] = -jnp.inf

    smax_ref[0] = jnp.maximum(smax_ref[0], jnp.max(a_src))
    smax_ref[1] = jnp.maximum(smax_ref[1], jnp.max(a_dst))

    @pl.when(i == GRID_TC - 1)
    def _():
        s = smax_ref[0] + smax_ref[1]
        m_ref[...] = jnp.zeros((1, 1), jnp.float32) + jnp.where(
            s > 0.0, s, 0.2 * s)


def _tc_first_body(x_ref, w_ref, as_ref, ad_ref, h_ref, asrc_ref, adst_ref,
                   m_ref, smax_ref):
    _tc_common(x_ref[...], pl.program_id(0), w_ref, as_ref, ad_ref, h_ref,
               asrc_ref, adst_ref, m_ref, smax_ref)


def _tc_mid_body(p_ref, dp_ref, b_ref, w_ref, as_ref, ad_ref, h_ref,
                 asrc_ref, adst_ref, m_ref, smax_ref):
    d = jnp.sum(dp_ref[...], axis=0)
    xin = p_ref[...] * (1.0 / (d + 1e-16))[:, None] + b_ref[...]
    xin = jnp.maximum(xin, 0.0)
    _tc_common(xin, pl.program_id(0), w_ref, as_ref, ad_ref, h_ref,
               asrc_ref, adst_ref, m_ref, smax_ref)


def _tc_layer(x_or_agg, dparts, b, W, a_s, a_d):
    if dparts is not None:
        body = _tc_mid_body
        in_specs = [
            pl.BlockSpec((R_TC, D), lambda i: (i, 0)),
            pl.BlockSpec((NS, R_TC), lambda i: (0, i)),
            pl.BlockSpec((1, D), lambda i: (0, 0)),
        ]
        args = (x_or_agg, dparts, b.reshape(1, D))
    else:
        body = _tc_first_body
        in_specs = [pl.BlockSpec((R_TC, D), lambda i: (i, 0))]
        args = (x_or_agg,)
    in_specs += [
        pl.BlockSpec((D, D), lambda i: (0, 0)),
        pl.BlockSpec((1, D), lambda i: (0, 0)),
        pl.BlockSpec((1, D), lambda i: (0, 0)),
    ]
    args = args + (W, a_s.reshape(1, D), a_d.reshape(1, D))

    return pl.pallas_call(
        body,
        grid=(GRID_TC,),
        in_specs=in_specs,
        out_specs=[
            pl.BlockSpec((R_TC, D), lambda i: (i, 0)),
            pl.BlockSpec((R_TC, 1), lambda i: (i, 0)),
            pl.BlockSpec((R_TC, 1), lambda i: (i, 0)),
            pl.BlockSpec((1, 1), lambda i: (0, 0)),
        ],
        out_shape=[
            jax.ShapeDtypeStruct((NP, D), jnp.float32),
            jax.ShapeDtypeStruct((NP, 1), jnp.float32),
            jax.ShapeDtypeStruct((NP, 1), jnp.float32),
            jax.ShapeDtypeStruct((1, 1), jnp.float32),
        ],
        scratch_shapes=[pltpu.SMEM((2,), jnp.float32)],
    )(*args)


def _tc_final_body(p_ref, dp_ref, b_ref, o_ref):
    d = jnp.sum(dp_ref[...], axis=0)
    o_ref[...] = p_ref[...] * (1.0 / (d + 1e-16))[:, None] + b_ref[...]


def _tc_final(agg, dparts, b):
    return pl.pallas_call(
        _tc_final_body,
        grid=(GRID_TC,),
        in_specs=[
            pl.BlockSpec((R_TC, D), lambda i: (i, 0)),
            pl.BlockSpec((NS, R_TC), lambda i: (0, i)),
            pl.BlockSpec((1, D), lambda i: (0, 0)),
        ],
        out_specs=pl.BlockSpec((R_TC, D), lambda i: (i, 0)),
        out_shape=jax.ShapeDtypeStruct((NP, D), jnp.float32),
    )(agg, dparts, b.reshape(1, D))


def _sc_body(ht_hbm, as_hbm, ad_hbm, m_hbm, src_hbm, dst_hbm,
             outp_hbm, dpart_hbm,
             dst2d, srcv0, srcv1, srcv2, srcv3, i2v0, i2v1, i2v2, i2v3, wv,
             rows0, rows1, rows2, rows3, asl, adl, dl, ml, osh,
             dsem0, dsem1, dsem2, dsem3, gsem0, gsem1, gsem2, gsem3,
             ssem0, ssem1, ssem2, ssem3):
    cid = lax.axis_index("c")
    sid = lax.axis_index("s")

    pltpu.sync_copy(as_hbm, asl)
    pltpu.sync_copy(ad_hbm, adl)
    pltpu.sync_copy(m_hbm, ml)
    pltpu.sync_copy(dst_hbm.at[sid], dst2d)

    z16 = jnp.zeros((16,), jnp.float32)

    @plsc.parallel_loop(0, NP // 16, 1, unroll=4)
    def _(i):
        dl[pl.ds(i * 16, 16)] = z16

    @plsc.parallel_loop(0, K, 1, unroll=4)
    def _(r):
        for cg in range(DH // 16):
            rows0[r, pl.ds(cg * 16, 16)] = z16

    row0 = sid * ROWS_PT
    for t in range(RCH):
        pltpu.sync_copy(rows0, osh.at[pl.ds(row0 + t * K, K)])
    plsc.subcore_barrier()

    mv = ml[...]
    base0 = sid * CPT
    rowss = (rows0, rows1, rows2, rows3)
    srcvs = (srcv0, srcv1, srcv2, srcv3)
    i2vs = (i2v0, i2v1, i2v2, i2v3)
    dsems = (dsem0, dsem1, dsem2, dsem3)
    gsems = (gsem0, gsem1, gsem2, gsem3)
    ssems = (ssem0, ssem1, ssem2, ssem3)

    def fetch_src(c, f):
        pltpu.async_copy(src_hbm.at[sid, c], srcvs[f], dsems[f])

    def prep_gather(c, g):
        pltpu.make_async_copy(src_hbm.at[sid, c], srcvs[g], dsems[g]).wait()
        for j in range(K // 16):
            sl = pl.ds(j * 16, 16)
            s1 = srcvs[g][sl]
            i2vs[g][sl] = s1 + s1 + cid
        pltpu.async_copy(ht_hbm.at[i2vs[g]], rowss[g], gsems[g])

    fetch_src(0, 0)
    fetch_src(1, 1)
    fetch_src(2, 2)
    prep_gather(0, 0)
    prep_gather(1, 1)

    def outer(ti, _):
        for b in range(NBUF):
            c = ti * NBUF + b
            g = (b + 2) % NBUF
            f = (b + 3) % NBUF
            for j in range(K // 16):
                sl = pl.ds(j * 16, 16)
                i_s = srcvs[b][sl]
                i_d = dst2d[c, sl]
                av = plsc.load_gather(asl, [i_s]) + plsc.load_gather(adl, [i_d])
                av = jnp.where(av >= 0.0, av, 0.2 * av)
                w = jnp.exp(av - mv)
                gid = base0 + c * K + j * 16 + lax.iota(jnp.int32, 16)
                w = jnp.where(gid < E_TOT, w, 0.0)
                wv[sl] = w
                plsc.addupdate_scatter(dl, [i_d], w)

            @pl.when(c + 3 < NCHUNK)
            def _():
                fetch_src(c + 3, f)

            @pl.when(c + 2 < NCHUNK)
            def _():
                @pl.when(c >= 2)
                def _():
                    pltpu.make_async_copy(
                        rowss[g], osh.at[dst2d.at[c - 2]], ssems[g]).wait()
                prep_gather(c + 2, g)

            pltpu.make_async_copy(ht_hbm.at[i2vs[b]], rowss[b],
                                  gsems[b]).wait()

            rb = rowss[b]

            @plsc.parallel_loop(0, K, 1, unroll=8)
            def _(r):
                wb = plsc.load_gather(wv, [jnp.zeros((16,), jnp.int32) + r])
                for cg in range(DH // 16):
                    sl2 = pl.ds(cg * 16, 16)
                    rb[r, sl2] = rb[r, sl2] * wb

            pltpu.async_copy(rb, osh.at[dst2d.at[c]], ssems[b], add=True)
        return 0
    lax.fori_loop(0, NCHUNK // NBUF, outer, 0)

    b2 = (NCHUNK - 2) % NBUF
    b1 = (NCHUNK - 1) % NBUF
    pltpu.make_async_copy(rowss[b2], osh.at[dst2d.at[NCHUNK - 2]],
                          ssems[b2]).wait()
    pltpu.make_async_copy(rowss[b1], osh.at[dst2d.at[NCHUNK - 1]],
                          ssems[b1]).wait()

    plsc.subcore_barrier()

    for t in range(RCH):
        sl = pl.ds(row0 + t * K, K)
        pltpu.sync_copy(osh.at[sl], rows0)
        pltpu.sync_copy(rows0, outp_hbm.at[sl, pl.ds(cid * DH, DH)])

    @pl.when(cid == 0)
    def _():
        pltpu.sync_copy(dl, dpart_hbm.at[sid])


def _sc_edge_pass(ht, a_src, a_dst, m16, srcp, dstp):
    mesh = plsc.VectorSubcoreMesh(core_axis_name="c", subcore_axis_name="s")
    kfn = pl.kernel(
        _sc_body,
        out_type=[
            jax.ShapeDtypeStruct((NP, D), jnp.float32),
            jax.ShapeDtypeStruct((NS, NP), jnp.float32),
        ],
        mesh=mesh,
        scratch_types=(
            [pltpu.VMEM((NCHUNK, K), jnp.int32)]
            + [pltpu.VMEM((K,), jnp.int32)] * 8
            + [pltpu.VMEM((K,), jnp.float32)]
            + [pltpu.VMEM((K, DH), jnp.float32)] * 4
            + [pltpu.VMEM((NP,), jnp.float32)] * 3
            + [pltpu.VMEM((16,), jnp.float32)]
            + [pltpu.VMEM_SHARED((NP, DH), jnp.float32)]
            + [pltpu.SemaphoreType.DMA] * 12
        ),
        compiler_params=pltpu.CompilerParams(needs_layout_passes=False,
                                             use_tc_tiling_on_sc=False),
    )
    return kfn(ht, a_src, a_dst, m16, srcp, dstp)


def kernel(x, edge_index, W1, as1, ad1, b1, W2, as2, ad2, b2, W3, as3, ad3, b3):
    n = x.shape[0]
    loops = jnp.arange(n, dtype=jnp.int32)
    zpad = jnp.zeros((E_PAD - E_TOT,), jnp.int32)
    srcp = jnp.concatenate(
        [edge_index[0].astype(jnp.int32), loops, zpad]).reshape(NS, NCHUNK, K)
    dstp = jnp.concatenate(
        [edge_index[1].astype(jnp.int32), loops, zpad]).reshape(NS, NCHUNK, K)
    xp = jnp.pad(x, ((0, NP - n), (0, 0)))

    h, a_s, a_d, m = _tc_layer(xp, None, None, W1, as1, ad1)
    agg, dparts = _sc_edge_pass(h.reshape(2 * NP, DH), a_s.reshape(NP),
                                a_d.reshape(NP), jnp.tile(m.reshape(1), 16),
                                srcp, dstp)
    h, a_s, a_d, m = _tc_layer(agg, dparts, b1, W2, as2, ad2)
    agg, dparts = _sc_edge_pass(h.reshape(2 * NP, DH), a_s.reshape(NP),
                                a_d.reshape(NP), jnp.tile(m.reshape(1), 16),
                                srcp, dstp)
    h, a_s, a_d, m = _tc_layer(agg, dparts, b2, W3, as3, ad3)
    agg, dparts = _sc_edge_pass(h.reshape(2 * NP, DH), a_s.reshape(NP),
                                a_d.reshape(NP), jnp.tile(m.reshape(1), 16),
                                srcp, dstp)
    out = _tc_final(agg, dparts, b3)
    return out[:n]


# speedup vs baseline: 1.4498x; 1.4498x over previous
import jax
import jax.numpy as jnp
from jax import lax
from jax.experimental import pallas as pl
from jax.experimental.pallas import tpu as pltpu
from jax.experimental.pallas import tpu_sc as plsc

N = 10000
NP = 10240
D = 128
DH = 64
E_TOT = 330000
NC = 2
NS = 16
K = 128
CPT = 20736
NCHUNK = CPT // K
E_PAD = NS * CPT
ROWS_PT = NP // NS
RCH = ROWS_PT // K
R_TC = 1024
GRID_TC = NP // R_TC


def _tc_common(xin, i, w_ref, as_ref, ad_ref, h_ref, asrc_ref, adst_ref,
               m_ref, smax_ref):
    h = jnp.dot(xin, w_ref[...], preferred_element_type=jnp.float32)
    h_ref[...] = h
    a_src = jnp.sum(h * as_ref[...], axis=1, keepdims=True)
    a_dst = jnp.sum(h * ad_ref[...], axis=1, keepdims=True)
    asrc_ref[...] = a_src
    adst_ref[...] = a_dst

    @pl.when(i == 0)
    def _():
        smax_ref[0] = -jnp.inf
        smax_ref[1] = -jnp.inf

    smax_ref[0] = jnp.maximum(smax_ref[0], jnp.max(a_src))
    smax_ref[1] = jnp.maximum(smax_ref[1], jnp.max(a_dst))

    @pl.when(i == GRID_TC - 1)
    def _():
        s = smax_ref[0] + smax_ref[1]
        m_ref[...] = jnp.zeros((1, 1), jnp.float32) + jnp.where(
            s > 0.0, s, 0.2 * s)


def _tc_first_body(x_ref, w_ref, as_ref, ad_ref, h_ref, asrc_ref, adst_ref,
                   m_ref, smax_ref):
    _tc_common(x_ref[...], pl.program_id(0), w_ref, as_ref, ad_ref, h_ref,
               asrc_ref, adst_ref, m_ref, smax_ref)


def _tc_mid_body(p_ref, dp_ref, b_ref, w_ref, as_ref, ad_ref, h_ref,
                 asrc_ref, adst_ref, m_ref, smax_ref):
    d = jnp.sum(dp_ref[...], axis=0)
    xin = p_ref[...] * (1.0 / (d + 1e-16))[:, None] + b_ref[...]
    xin = jnp.maximum(xin, 0.0)
    _tc_common(xin, pl.program_id(0), w_ref, as_ref, ad_ref, h_ref,
               asrc_ref, adst_ref, m_ref, smax_ref)


def _tc_layer(x_or_agg, dparts, b, W, a_s, a_d):
    if dparts is not None:
        body = _tc_mid_body
        in_specs = [
            pl.BlockSpec((R_TC, D), lambda i: (i, 0)),
            pl.BlockSpec((NS, R_TC), lambda i: (0, i)),
            pl.BlockSpec((1, D), lambda i: (0, 0)),
        ]
        args = (x_or_agg, dparts, b.reshape(1, D))
    else:
        body = _tc_first_body
        in_specs = [pl.BlockSpec((R_TC, D), lambda i: (i, 0))]
        args = (x_or_agg,)
    in_specs += [
        pl.BlockSpec((D, D), lambda i: (0, 0)),
        pl.BlockSpec((1, D), lambda i: (0, 0)),
        pl.BlockSpec((1, D), lambda i: (0, 0)),
    ]
    args = args + (W, a_s.reshape(1, D), a_d.reshape(1, D))

    return pl.pallas_call(
        body,
        grid=(GRID_TC,),
        in_specs=in_specs,
        out_specs=[
            pl.BlockSpec((R_TC, D), lambda i: (i, 0)),
            pl.BlockSpec((R_TC, 1), lambda i: (i, 0)),
            pl.BlockSpec((R_TC, 1), lambda i: (i, 0)),
            pl.BlockSpec((1, 1), lambda i: (0, 0)),
        ],
        out_shape=[
            jax.ShapeDtypeStruct((NP, D), jnp.float32),
            jax.ShapeDtypeStruct((NP, 1), jnp.float32),
            jax.ShapeDtypeStruct((NP, 1), jnp.float32),
            jax.ShapeDtypeStruct((1, 1), jnp.float32),
        ],
        scratch_shapes=[pltpu.SMEM((2,), jnp.float32)],
    )(*args)


def _tc_final_body(p_ref, dp_ref, b_ref, o_ref):
    d = jnp.sum(dp_ref[...], axis=0)
    o_ref[...] = p_ref[...] * (1.0 / (d + 1e-16))[:, None] + b_ref[...]


def _tc_final(agg, dparts, b):
    return pl.pallas_call(
        _tc_final_body,
        grid=(GRID_TC,),
        in_specs=[
            pl.BlockSpec((R_TC, D), lambda i: (i, 0)),
            pl.BlockSpec((NS, R_TC), lambda i: (0, i)),
            pl.BlockSpec((1, D), lambda i: (0, 0)),
        ],
        out_specs=pl.BlockSpec((R_TC, D), lambda i: (i, 0)),
        out_shape=jax.ShapeDtypeStruct((NP, D), jnp.float32),
    )(agg, dparts, b.reshape(1, D))


def _sc_body(ht_hbm, as_hbm, ad_hbm, m_hbm, src_hbm, dst_hbm,
             outp_hbm, dpart_hbm,
             src2d, dst2d, i2v0, i2v1, wv, rows0, rows1,
             asl, adl, dl, ml, osh, gsem0, gsem1, ssem0, ssem1):
    cid = lax.axis_index("c")
    sid = lax.axis_index("s")

    pltpu.sync_copy(as_hbm, asl)
    pltpu.sync_copy(ad_hbm, adl)
    pltpu.sync_copy(m_hbm, ml)
    pltpu.sync_copy(src_hbm.at[sid], src2d)
    pltpu.sync_copy(dst_hbm.at[sid], dst2d)

    z16 = jnp.zeros((16,), jnp.float32)

    @plsc.parallel_loop(0, NP // 16, 1, unroll=4)
    def _(i):
        dl[pl.ds(i * 16, 16)] = z16

    @plsc.parallel_loop(0, K, 1, unroll=4)
    def _(r):
        for cg in range(DH // 16):
            rows0[r, pl.ds(cg * 16, 16)] = z16

    row0 = sid * ROWS_PT
    for t in range(RCH):
        pltpu.sync_copy(rows0, osh.at[pl.ds(row0 + t * K, K)])
    plsc.subcore_barrier()

    mv = ml[...]
    base0 = sid * CPT
    rows = (rows0, rows1)
    i2vs = (i2v0, i2v1)
    gsems = (gsem0, gsem1)
    ssems = (ssem0, ssem1)

    for j in range(K // 16):
        sl = pl.ds(j * 16, 16)
        s0 = src2d[0, sl]
        i2v0[sl] = s0 + s0 + cid
    pltpu.async_copy(ht_hbm.at[i2v0], rows0, gsem0)

    def outer(ti, _):
        for b in range(2):
            c = ti * 2 + b
            ob = 1 - b
            for j in range(K // 16):
                sl = pl.ds(j * 16, 16)
                i_s = src2d[c, sl]
                i_d = dst2d[c, sl]
                av = plsc.load_gather(asl, [i_s]) + plsc.load_gather(adl, [i_d])
                av = jnp.where(av >= 0.0, av, 0.2 * av)
                w = jnp.exp(av - mv)
                gid = base0 + c * K + j * 16 + lax.iota(jnp.int32, 16)
                w = jnp.where(gid < E_TOT, w, 0.0)
                wv[sl] = w
                plsc.addupdate_scatter(dl, [i_d], w)

            @pl.when(c + 1 < NCHUNK)
            def _():
                @pl.when(c >= 1)
                def _():
                    pltpu.make_async_copy(
                        rows[ob], osh.at[dst2d.at[c - 1]], ssems[ob]).wait()
                for j in range(K // 16):
                    sl = pl.ds(j * 16, 16)
                    s1 = src2d[c + 1, sl]
                    i2vs[ob][sl] = s1 + s1 + cid
                pltpu.async_copy(ht_hbm.at[i2vs[ob]], rows[ob], gsems[ob])

            pltpu.make_async_copy(ht_hbm.at[i2vs[b]], rows[b],
                                  gsems[b]).wait()

            rb = rows[b]

            @plsc.parallel_loop(0, K, 1, unroll=8)
            def _(r):
                wb = plsc.load_gather(wv, [jnp.zeros((16,), jnp.int32) + r])
                for cg in range(DH // 16):
                    sl2 = pl.ds(cg * 16, 16)
                    rb[r, sl2] = rb[r, sl2] * wb

            pltpu.async_copy(rb, osh.at[dst2d.at[c]], ssems[b], add=True)
        return 0
    lax.fori_loop(0, NCHUNK // 2, outer, 0)

    pltpu.make_async_copy(rows0, osh.at[dst2d.at[NCHUNK - 2]], ssem0).wait()
    pltpu.make_async_copy(rows1, osh.at[dst2d.at[NCHUNK - 1]], ssem1).wait()

    plsc.subcore_barrier()

    for t in range(RCH):
        sl = pl.ds(row0 + t * K, K)
        pltpu.sync_copy(osh.at[sl], rows0)
        pltpu.sync_copy(rows0, outp_hbm.at[sl, pl.ds(cid * DH, DH)])

    @pl.when(cid == 0)
    def _():
        pltpu.sync_copy(dl, dpart_hbm.at[sid])


def _sc_edge_pass(ht, a_src, a_dst, m16, srcp, dstp):
    mesh = plsc.VectorSubcoreMesh(core_axis_name="c", subcore_axis_name="s")
    kfn = pl.kernel(
        _sc_body,
        out_type=[
            jax.ShapeDtypeStruct((NP, D), jnp.float32),
            jax.ShapeDtypeStruct((NS, NP), jnp.float32),
        ],
        mesh=mesh,
        scratch_types=(
            [pltpu.VMEM((NCHUNK, K), jnp.int32)] * 2
            + [pltpu.VMEM((K,), jnp.int32)] * 2
            + [pltpu.VMEM((K,), jnp.float32)]
            + [pltpu.VMEM((K, DH), jnp.float32)] * 2
            + [pltpu.VMEM((NP,), jnp.float32)] * 3
            + [pltpu.VMEM((16,), jnp.float32)]
            + [pltpu.VMEM_SHARED((NP, DH), jnp.float32)]
            + [pltpu.SemaphoreType.DMA] * 4
        ),
        compiler_params=pltpu.CompilerParams(needs_layout_passes=False,
                                             use_tc_tiling_on_sc=False),
    )
    return kfn(ht, a_src, a_dst, m16, srcp, dstp)


def kernel(x, edge_index, W1, as1, ad1, b1, W2, as2, ad2, b2, W3, as3, ad3, b3):
    n = x.shape[0]
    loops = jnp.arange(n, dtype=jnp.int32)
    zpad = jnp.zeros((E_PAD - E_TOT,), jnp.int32)
    srcp = jnp.concatenate(
        [edge_index[0].astype(jnp.int32), loops, zpad]).reshape(NS, NCHUNK, K)
    dstp = jnp.concatenate(
        [edge_index[1].astype(jnp.int32), loops, zpad]).reshape(NS, NCHUNK, K)
    xp = jnp.pad(x, ((0, NP - n), (0, 0)))

    h, a_s, a_d, m = _tc_layer(xp, None, None, W1, as1, ad1)
    agg, dparts = _sc_edge_pass(h.reshape(2 * NP, DH), a_s.reshape(NP),
                                a_d.reshape(NP), jnp.tile(m.reshape(1), 16),
                                srcp, dstp)
    h, a_s, a_d, m = _tc_layer(agg, dparts, b1, W2, as2, ad2)
    agg, dparts = _sc_edge_pass(h.reshape(2 * NP, DH), a_s.reshape(NP),
                                a_d.reshape(NP), jnp.tile(m.reshape(1), 16),
                                srcp, dstp)
    h, a_s, a_d, m = _tc_layer(agg, dparts, b2, W3, as3, ad3)
    agg, dparts = _sc_edge_pass(h.reshape(2 * NP, DH), a_s.reshape(NP),
                                a_d.reshape(NP), jnp.tile(m.reshape(1), 16),
                                srcp, dstp)
    out = _tc_final(agg, dparts, b3)
    return out[:n]


# speedup vs baseline: 1.7591x; 1.2134x over previous
import jax
import jax.numpy as jnp
import numpy as np
from jax import lax
from jax.experimental import pallas as pl
from jax.experimental.pallas import tpu as pltpu
from jax.experimental.pallas import tpu_sc as plsc

N = 10000
NP = 10240
D = 128
DH = 64
E_TOT = 330000
NC = 2
NS = 16
K = 128
CPT = 20736
NCHUNK = CPT // K
E_PAD = NS * CPT
ROWS_PT = NP // NS
RCH = ROWS_PT // K
R_TC = 1024
GRID_TC = NP // R_TC

_pb = np.arange(32).reshape(2, 16).T.reshape(32)
PERM = np.concatenate([32 * j + _pb for j in range(4)])


def _tc_common(xin, i, w_ref, as_ref, ad_ref, ht_ref, asrc_ref, adst_ref,
               m_ref, smax_ref):
    h = jnp.dot(xin, w_ref[...], preferred_element_type=jnp.float32)
    ht_ref[...] = h.astype(jnp.bfloat16)
    a_src = jnp.sum(h * as_ref[...], axis=1, keepdims=True)
    a_dst = jnp.sum(h * ad_ref[...], axis=1, keepdims=True)
    asrc_ref[...] = a_src
    adst_ref[...] = a_dst

    @pl.when(i == 0)
    def _():
        smax_ref[0] = -jnp.inf
        smax_ref[1] = -jnp.inf

    smax_ref[0] = jnp.maximum(smax_ref[0], jnp.max(a_src))
    smax_ref[1] = jnp.maximum(smax_ref[1], jnp.max(a_dst))

    @pl.when(i == GRID_TC - 1)
    def _():
        s = smax_ref[0] + smax_ref[1]
        m_ref[...] = jnp.zeros((1, 1), jnp.float32) + jnp.where(
            s > 0.0, s, 0.2 * s)


def _tc_first_body(x_ref, w_ref, as_ref, ad_ref, ht_ref, asrc_ref, adst_ref,
                   m_ref, smax_ref):
    _tc_common(x_ref[...], pl.program_id(0), w_ref, as_ref, ad_ref, ht_ref,
               asrc_ref, adst_ref, m_ref, smax_ref)


def _tc_mid_body(p_ref, dp_ref, b_ref, w_ref, as_ref, ad_ref, ht_ref,
                 asrc_ref, adst_ref, m_ref, smax_ref):
    d = jnp.sum(dp_ref[...], axis=0)
    xin = p_ref[...] * (1.0 / (d + 1e-16))[:, None] + b_ref[...]
    xin = jnp.maximum(xin, 0.0)
    _tc_common(xin, pl.program_id(0), w_ref, as_ref, ad_ref, ht_ref,
               asrc_ref, adst_ref, m_ref, smax_ref)


def _tc_layer(x_or_agg, dparts, b, W, a_s, a_d):
    if dparts is not None:
        body = _tc_mid_body
        in_specs = [
            pl.BlockSpec((R_TC, D), lambda i: (i, 0)),
            pl.BlockSpec((NS, R_TC), lambda i: (0, i)),
            pl.BlockSpec((1, D), lambda i: (0, 0)),
        ]
        args = (x_or_agg, dparts, b.reshape(1, D))
    else:
        body = _tc_first_body
        in_specs = [pl.BlockSpec((R_TC, D), lambda i: (i, 0))]
        args = (x_or_agg,)
    in_specs += [
        pl.BlockSpec((D, D), lambda i: (0, 0)),
        pl.BlockSpec((1, D), lambda i: (0, 0)),
        pl.BlockSpec((1, D), lambda i: (0, 0)),
    ]
    args = args + (W, a_s.reshape(1, D), a_d.reshape(1, D))

    return pl.pallas_call(
        body,
        grid=(GRID_TC,),
        in_specs=in_specs,
        out_specs=[
            pl.BlockSpec((R_TC, D), lambda i: (i, 0)),
            pl.BlockSpec((R_TC, 1), lambda i: (i, 0)),
            pl.BlockSpec((R_TC, 1), lambda i: (i, 0)),
            pl.BlockSpec((1, 1), lambda i: (0, 0)),
        ],
        out_shape=[
            jax.ShapeDtypeStruct((NP, D), jnp.bfloat16),
            jax.ShapeDtypeStruct((NP, 1), jnp.float32),
            jax.ShapeDtypeStruct((NP, 1), jnp.float32),
            jax.ShapeDtypeStruct((1, 1), jnp.float32),
        ],
        scratch_shapes=[pltpu.SMEM((2,), jnp.float32)],
    )(*args)


def _tc_final_body(p_ref, dp_ref, b_ref, o_ref):
    d = jnp.sum(dp_ref[...], axis=0)
    o_ref[...] = p_ref[...] * (1.0 / (d + 1e-16))[:, None] + b_ref[...]


def _tc_final(agg, dparts, b):
    return pl.pallas_call(
        _tc_final_body,
        grid=(GRID_TC,),
        in_specs=[
            pl.BlockSpec((R_TC, D), lambda i: (i, 0)),
            pl.BlockSpec((NS, R_TC), lambda i: (0, i)),
            pl.BlockSpec((1, D), lambda i: (0, 0)),
        ],
        out_specs=pl.BlockSpec((R_TC, D), lambda i: (i, 0)),
        out_shape=jax.ShapeDtypeStruct((NP, D), jnp.float32),
    )(agg, dparts, b.reshape(1, D))


def _sc_body(ht_hbm, as_hbm, ad_hbm, m_hbm, src_hbm, dst_hbm,
             outp_hbm, dpart_hbm,
             src2d, dst2d, i2v0, i2v1, wv, rows0, rows1, sbuf,
             asl, adl, dl, ml, osh, gsem0, gsem1, ssem):
    cid = lax.axis_index("c")
    sid = lax.axis_index("s")

    pltpu.sync_copy(as_hbm, asl)
    pltpu.sync_copy(ad_hbm, adl)
    pltpu.sync_copy(m_hbm, ml)
    pltpu.sync_copy(src_hbm.at[sid], src2d)
    pltpu.sync_copy(dst_hbm.at[sid], dst2d)

    z16 = jnp.zeros((16,), jnp.float32)

    @plsc.parallel_loop(0, NP // 16, 1, unroll=4)
    def _(i):
        dl[pl.ds(i * 16, 16)] = z16

    @plsc.parallel_loop(0, K, 1, unroll=4)
    def _(r):
        for cg in range(DH // 16):
            sbuf[r, pl.ds(cg * 16, 16)] = z16

    row0 = sid * ROWS_PT
    for t in range(RCH):
        pltpu.sync_copy(sbuf, osh.at[pl.ds(row0 + t * K, K)])
    plsc.subcore_barrier()

    mv = ml[...]
    base0 = sid * CPT
    rows = (rows0, rows1)
    i2vs = (i2v0, i2v1)
    gsems = (gsem0, gsem1)

    for j in range(K // 16):
        sl = pl.ds(j * 16, 16)
        s0 = src2d[0, sl]
        i2v0[sl] = s0 + s0 + cid
    pltpu.async_copy(ht_hbm.at[i2v0], rows0, gsem0)

    def outer(ti, _):
        for b in range(2):
            c = ti * 2 + b
            ob = 1 - b
            for j in range(K // 16):
                sl = pl.ds(j * 16, 16)
                i_s = src2d[c, sl]
                i_d = dst2d[c, sl]
                av = plsc.load_gather(asl, [i_s]) + plsc.load_gather(adl, [i_d])
                av = jnp.where(av >= 0.0, av, 0.2 * av)
                w = jnp.exp(av - mv)
                gid = base0 + c * K + j * 16 + lax.iota(jnp.int32, 16)
                w = jnp.where(gid < E_TOT, w, 0.0)
                wv[sl] = w
                plsc.addupdate_scatter(dl, [i_d], w)

            @pl.when(c + 1 < NCHUNK)
            def _():
                for j in range(K // 16):
                    sl = pl.ds(j * 16, 16)
                    s1 = src2d[c + 1, sl]
                    i2vs[ob][sl] = s1 + s1 + cid
                pltpu.async_copy(ht_hbm.at[i2vs[ob]], rows[ob], gsems[ob])

            pltpu.make_async_copy(ht_hbm.at[i2vs[b]], rows[b],
                                  gsems[b]).wait()

            @pl.when(c >= 1)
            def _():
                pltpu.make_async_copy(sbuf, osh.at[dst2d.at[c - 1]],
                                      ssem).wait()

            rb = rows[b]

            @plsc.parallel_loop(0, K, 1, unroll=8)
            def _(r):
                wb = plsc.load_gather(wv, [jnp.zeros((16,), jnp.int32) + r])
                for half in range(2):
                    x32 = rb[r, pl.ds(half * 32, 32)]
                    lo, hi = plsc.unpack(x32,
                                         format=plsc.PackFormat.INTERLEAVED)
                    sbuf[r, pl.ds(half * 32, 16)] = lo * wb
                    sbuf[r, pl.ds(half * 32 + 16, 16)] = hi * wb

            pltpu.async_copy(sbuf, osh.at[dst2d.at[c]], ssem, add=True)
        return 0
    lax.fori_loop(0, NCHUNK // 2, outer, 0)

    pltpu.make_async_copy(sbuf, osh.at[dst2d.at[NCHUNK - 1]], ssem).wait()

    plsc.subcore_barrier()

    for t in range(RCH):
        sl = pl.ds(row0 + t * K, K)
        pltpu.sync_copy(osh.at[sl], sbuf)
        pltpu.sync_copy(sbuf, outp_hbm.at[sl, pl.ds(cid * DH, DH)])

    @pl.when(cid == 0)
    def _():
        pltpu.sync_copy(dl, dpart_hbm.at[sid])


def _sc_edge_pass(ht, a_src, a_dst, m16, srcp, dstp):
    mesh = plsc.VectorSubcoreMesh(core_axis_name="c", subcore_axis_name="s")
    kfn = pl.kernel(
        _sc_body,
        out_type=[
            jax.ShapeDtypeStruct((NP, D), jnp.float32),
            jax.ShapeDtypeStruct((NS, NP), jnp.float32),
        ],
        mesh=mesh,
        scratch_types=(
            [pltpu.VMEM((NCHUNK, K), jnp.int32)] * 2
            + [pltpu.VMEM((K,), jnp.int32)] * 2
            + [pltpu.VMEM((K,), jnp.float32)]
            + [pltpu.VMEM((K, DH), jnp.bfloat16)] * 2
            + [pltpu.VMEM((K, DH), jnp.float32)]
            + [pltpu.VMEM((NP,), jnp.float32)] * 3
            + [pltpu.VMEM((16,), jnp.float32)]
            + [pltpu.VMEM_SHARED((NP, DH), jnp.float32)]
            + [pltpu.SemaphoreType.DMA] * 3
        ),
        compiler_params=pltpu.CompilerParams(needs_layout_passes=False,
                                             use_tc_tiling_on_sc=False),
    )
    return kfn(ht, a_src, a_dst, m16, srcp, dstp)


def kernel(x, edge_index, W1, as1, ad1, b1, W2, as2, ad2, b2, W3, as3, ad3, b3):
    n = x.shape[0]
    loops = jnp.arange(n, dtype=jnp.int32)
    zpad = jnp.zeros((E_PAD - E_TOT,), jnp.int32)
    srcp = jnp.concatenate(
        [edge_index[0].astype(jnp.int32), loops, zpad]).reshape(NS, NCHUNK, K)
    dstp = jnp.concatenate(
        [edge_index[1].astype(jnp.int32), loops, zpad]).reshape(NS, NCHUNK, K)
    xp = jnp.pad(x, ((0, NP - n), (0, 0)))

    layers = [
        (W1[:, PERM], as1[PERM], ad1[PERM], b1),
        (W2[:, PERM], as2[PERM], ad2[PERM], b2),
        (W3[:, PERM], as3[PERM], ad3[PERM], b3),
    ]

    agg, dparts = None, None
    for li, (W, a_s, a_d, b_prev) in enumerate(layers):
        prev_b = layers[li - 1][3] if li > 0 else None
        ht, a_sv, a_dv, m = _tc_layer(xp if li == 0 else agg, dparts,
                                      prev_b, W, a_s, a_d)
        agg, dparts = _sc_edge_pass(ht.reshape(2 * NP, DH),
                                    a_sv.reshape(NP), a_dv.reshape(NP),
                                    jnp.tile(m.reshape(1), 16), srcp, dstp)
    out = _tc_final(agg, dparts, b3)
    return out[:n]
